# Initial kernel scaffold; baseline (speedup 1.0000x reference)
import jax
import jax.numpy as jnp
from jax.experimental import pallas as pl


def kernel(x, edge_index, causal_edge_index, W_high_temp, b_high_temp, W_low_temp, b_low_temp, W_hg_nei, W_hg_self, b_hg, W_lg_nei, W_lg_self, b_lg, W_lc, W_hr, b_hr, W_lr, b_lr, g_h, be_h, g_l, be_l, W_att, b_att, W_gr, b_gr, g_g, be_g):
    raise NotImplementedError("write your pallas kernel here")


# speedup vs baseline: 35.3163x; 35.3163x over previous
import functools

import jax
import jax.numpy as jnp
from jax import lax
from jax.experimental import pallas as pl
from jax.experimental.pallas import tpu as pltpu
from jax.experimental.pallas import tpu_sc as plsc

_NC = 2
_NS = 16
_NW = _NC * _NS
_CHUNK_ROWS = 4
_EDGES_PER_STEP = _CHUNK_ROWS * 128


def _ceil_to(v, m):
    return (v + m - 1) // m * m


def _sc_aggregate(xp, srcr, dstr, csrcr, cdstr, zrows, R, CW, n_base, n_caus):
    rows_sub = R // _NS
    base_wrows = n_base * _CHUNK_ROWS
    caus_wrows = n_caus * _CHUNK_ROWS

    mesh = plsc.VectorSubcoreMesh(
        core_axis_name="c", subcore_axis_name="s",
        num_cores=_NC, num_subcores=_NS)

    @functools.partial(
        pl.kernel,
        out_type=(jax.ShapeDtypeStruct((_NC, R, CW), jnp.float32),
                  jax.ShapeDtypeStruct((_NC, R, CW), jnp.float32)),
        mesh=mesh,
        compiler_params=pltpu.CompilerParams(use_tc_tiling_on_sc=False),
        scratch_types=(
            pltpu.VMEM_SHARED((R, CW), jnp.float32),
            pltpu.VMEM_SHARED((R, CW), jnp.float32),
            pltpu.VMEM((_CHUNK_ROWS, 128), jnp.int32),
            pltpu.VMEM((_CHUNK_ROWS, 128), jnp.int32),
            pltpu.VMEM((_CHUNK_ROWS, 128, CW), jnp.float32),
        ),
    )
    def k(xp_h, src_h, dst_h, csrc_h, cdst_h, z_h, agg_h, cagg_h,
          acc, cacc, sv, dv, rows):
        c = lax.axis_index("c")
        s = lax.axis_index("s")
        w = s * _NC + c

        pltpu.sync_copy(z_h, acc.at[pl.ds(s * rows_sub, rows_sub)])
        pltpu.sync_copy(z_h, cacc.at[pl.ds(s * rows_sub, rows_sub)])
        plsc.subcore_barrier()

        def edge_pass(src_ref, dst_ref, acc_ref, n_steps, wrows):
            base_row = w * wrows

            def step(i, carry):
                r0 = base_row + i * _CHUNK_ROWS
                pltpu.sync_copy(src_ref.at[pl.ds(r0, _CHUNK_ROWS)], sv)
                pltpu.sync_copy(dst_ref.at[pl.ds(r0, _CHUNK_ROWS)], dv)
                for j in range(_CHUNK_ROWS):
                    pltpu.sync_copy(xp_h.at[sv.at[j]], rows.at[j])
                for j in range(_CHUNK_ROWS):
                    pltpu.sync_copy(rows.at[j], acc_ref.at[dv.at[j]], add=True)
                return carry

            lax.fori_loop(0, n_steps, step, 0)

        edge_pass(src_h, dst_h, acc, n_base, base_wrows)
        edge_pass(csrc_h, cdst_h, cacc, n_caus, caus_wrows)
        plsc.subcore_barrier()

        sl = pl.ds(s * rows_sub, rows_sub)
        pltpu.sync_copy(acc.at[sl], agg_h.at[c, sl])
        pltpu.sync_copy(cacc.at[sl], cagg_h.at[c, sl])

    return k(xp, srcr, dstr, csrcr, cdstr, zrows)


def _tc_dense(x2d, aggp, caggp, Wm, P, B, T, BT, OD, R, Nb):
    grid = (R // Nb,)

    def body(x_ref, ap_ref, cp_ref, w_ref, p_ref, f_ref, h_ref, l_ref):
        agg = ap_ref[0] + ap_ref[1]
        cagg = cp_ref[0] + cp_ref[1]
        aggT = agg.T
        caggT = cagg.T
        deg = aggT[BT:BT + 1, :]
        cdeg = caggT[BT:BT + 1, :]
        dmask = (deg > 0).astype(jnp.float32)
        cmask = (cdeg > 0).astype(jnp.float32)
        dinv = 1.0 / jnp.maximum(deg, 1.0)
        cinv = 1.0 / jnp.maximum(cdeg, 1.0)

        def mm(Wmat, Xmat):
            return lax.dot_general(
                Wmat, Xmat, (((1,), (0,)), ((), ())),
                precision=lax.Precision.HIGHEST,
                preferred_element_type=jnp.float32)

        def ln(h, gcol, bcol):
            mu = jnp.mean(h, axis=0, keepdims=True)
            xc = h - mu
            var = jnp.mean(xc * xc, axis=0, keepdims=True)
            return gcol * xc * lax.rsqrt(var + 1e-5) + bcol

        pcol = lambda k: p_ref[:, k:k + 1]
        batt = p_ref[0:1, 14:15]

        for b in range(B):
            xb = x_ref[b * T:(b + 1) * T, :]
            maT = aggT[b * T:(b + 1) * T, :] * dinv
            caT = caggT[b * T:(b + 1) * T, :] * cinv

            hp = (mm(w_ref[0], maT) + mm(w_ref[1], xb)
                  + pcol(0) * dmask + pcol(1))
            hn = ln(hp, pcol(6), pcol(7))
            high = jnp.where(hn > 0, hn, 0.1 * hn)

            lp = (mm(w_ref[2], maT) + mm(w_ref[3], caT) + mm(w_ref[4], xb)
                  + pcol(2) * dmask + pcol(3) * cmask + pcol(4))
            lnl = ln(lp, pcol(8), pcol(9))
            low = 0.5 * lnl * (1.0 + jnp.tanh(
                0.7978845608028654 * (lnl + 0.044715 * lnl * lnl * lnl)))

            res = mm(w_ref[5], 2.0 * xb) + pcol(5)
            res = ln(res, pcol(10), pcol(11))

            s = (jnp.sum(high * pcol(12), axis=0, keepdims=True)
                 + jnp.sum(low * pcol(13), axis=0, keepdims=True) + batt)
            alpha = 1.0 / (1.0 + jnp.exp(-s))
            fused = (alpha + 0.3) * high + (1.3 - alpha) * low + 0.1 * res

            f_ref[b] = fused
            h_ref[b] = high
            l_ref[b] = low

    CW = aggp.shape[2]
    out_sd = jax.ShapeDtypeStruct((B, OD, R), jnp.float32)
    return pl.pallas_call(
        body,
        grid=grid,
        in_specs=[
            pl.BlockSpec((BT, Nb), lambda i: (0, i)),
            pl.BlockSpec((_NC, Nb, CW), lambda i: (0, i, 0)),
            pl.BlockSpec((_NC, Nb, CW), lambda i: (0, i, 0)),
            pl.BlockSpec((6, OD, T), lambda i: (0, 0, 0)),
            pl.BlockSpec((OD, 16), lambda i: (0, 0)),
        ],
        out_specs=[
            pl.BlockSpec((B, OD, Nb), lambda i: (0, 0, i)),
            pl.BlockSpec((B, OD, Nb), lambda i: (0, 0, i)),
            pl.BlockSpec((B, OD, Nb), lambda i: (0, 0, i)),
        ],
        out_shape=(out_sd, out_sd, out_sd),
    )(x2d, aggp, caggp, Wm, P)


def kernel(x, edge_index, causal_edge_index, W_high_temp, b_high_temp,
           W_low_temp, b_low_temp, W_hg_nei, W_hg_self, b_hg, W_lg_nei,
           W_lg_self, b_lg, W_lc, W_hr, b_hr, W_lr, b_lr, g_h, be_h, g_l,
           be_l, W_att, b_att, W_gr, b_gr, g_g, be_g):
    B, T, N = x.shape
    E = edge_index.shape[1]
    EC = causal_edge_index.shape[1]
    OD = W_hg_nei.shape[1]
    BT = B * T
    CW = _ceil_to(BT + 1, 16)
    R = _ceil_to(N + 1, _NS * _EDGES_PER_STEP // 16)
    if R % (_NS * 8):
        R = _ceil_to(R, _NS * 8)
    Nb = 1024 if R % 1024 == 0 else 512

    f32 = jnp.float32

    x2d = x.reshape(BT, N)
    x2dp = jnp.pad(x2d, ((0, 0), (0, R - N)))
    xp = jnp.concatenate(
        [x2d.T, jnp.ones((N, 1), f32), jnp.zeros((N, CW - BT - 1), f32)],
        axis=1)

    def pack_edges(ei, ne):
        epad = _ceil_to(ne, _NW * _EDGES_PER_STEP)
        pe = epad - ne
        s = jnp.concatenate([ei[0], jnp.zeros((pe,), jnp.int32)])
        d = jnp.concatenate([ei[1], jnp.full((pe,), N, jnp.int32)])
        return (s.reshape(epad // 128, 128), d.reshape(epad // 128, 128),
                epad // (_NW * _EDGES_PER_STEP))

    srcr, dstr, n_base = pack_edges(edge_index, E)
    csrcr, cdstr, n_caus = pack_edges(causal_edge_index, EC)
    zrows = jnp.zeros((R // _NS, CW), f32)

    Ah = W_high_temp @ W_hg_nei
    Ch = W_high_temp @ (W_hg_self + 0.2 * W_hr)
    bAh = b_high_temp @ W_hg_nei
    ch = b_high_temp @ (W_hg_self + 0.2 * W_hr) + b_hg + 0.2 * b_hr
    Al = W_low_temp @ W_lg_nei
    Alc = W_low_temp @ W_lc
    Cl = W_low_temp @ (W_lg_self + 0.2 * W_lr)
    bAl = b_low_temp @ W_lg_nei
    bAlc = b_low_temp @ W_lc
    cl = b_low_temp @ (W_lg_self + 0.2 * W_lr) + b_lg + 0.2 * b_lr

    Wm = jnp.stack([Ah.T, Ch.T, Al.T, Alc.T, Cl.T, W_gr.T])
    P = jnp.stack(
        [bAh, ch, bAl, bAlc, cl, b_gr, g_h, be_h, g_l, be_l, g_g, be_g,
         W_att[:OD], W_att[OD:], jnp.full((OD,), b_att, f32),
         jnp.zeros((OD,), f32)], axis=1)

    aggp, caggp = _sc_aggregate(xp, srcr, dstr, csrcr, cdstr, zrows,
                                R, CW, n_base, n_caus)
    fused, high, low = _tc_dense(x2dp, aggp, caggp, Wm, P,
                                 B, T, BT, OD, R, Nb)
    return (fused[:, :, :N], high[:, :, :N], low[:, :, :N])


# speedup vs baseline: 38.0785x; 1.0782x over previous
import functools

import jax
import jax.numpy as jnp
from jax import lax
from jax.experimental import pallas as pl
from jax.experimental.pallas import tpu as pltpu
from jax.experimental.pallas import tpu_sc as plsc

_NC = 2
_NS = 16
_NW = _NC * _NS
_CHUNK_ROWS = 4
_EDGES_PER_STEP = _CHUNK_ROWS * 128


def _ceil_to(v, m):
    return (v + m - 1) // m * m


def _sc_aggregate(xp, srcr, dstr, csrcr, cdstr, zrows, R, CW, n_base, n_caus):
    rows_sub = R // _NS
    base_wrows = n_base * _CHUNK_ROWS
    caus_wrows = n_caus * _CHUNK_ROWS

    mesh = plsc.VectorSubcoreMesh(
        core_axis_name="c", subcore_axis_name="s",
        num_cores=_NC, num_subcores=_NS)

    @functools.partial(
        pl.kernel,
        out_type=(jax.ShapeDtypeStruct((_NC, R, CW), jnp.float32),
                  jax.ShapeDtypeStruct((_NC, R, CW), jnp.float32)),
        mesh=mesh,
        compiler_params=pltpu.CompilerParams(use_tc_tiling_on_sc=False),
        scratch_types=(
            pltpu.VMEM_SHARED((R, CW), jnp.float32),
            pltpu.VMEM_SHARED((R, CW), jnp.float32),
            pltpu.VMEM((_CHUNK_ROWS, 128), jnp.int32),
            pltpu.VMEM((_CHUNK_ROWS, 128), jnp.int32),
            pltpu.VMEM((_CHUNK_ROWS, 128, CW), jnp.float32),
            pltpu.SemaphoreType.DMA,
            pltpu.SemaphoreType.DMA,
        ),
    )
    def k(xp_h, src_h, dst_h, csrc_h, cdst_h, z_h, agg_h, cagg_h,
          acc, cacc, sv, dv, rows, gsem, ssem):
        c = lax.axis_index("c")
        s = lax.axis_index("s")
        w = s * _NC + c

        pltpu.sync_copy(z_h, acc.at[pl.ds(s * rows_sub, rows_sub)])
        pltpu.sync_copy(z_h, cacc.at[pl.ds(s * rows_sub, rows_sub)])
        plsc.subcore_barrier()

        def edge_pass(src_ref, dst_ref, acc_ref, n_steps, wrows):
            base_row = w * wrows

            def step(i, carry):
                r0 = base_row + i * _CHUNK_ROWS
                pltpu.sync_copy(src_ref.at[pl.ds(r0, _CHUNK_ROWS)], sv)
                pltpu.sync_copy(dst_ref.at[pl.ds(r0, _CHUNK_ROWS)], dv)
                gh = [pltpu.async_copy(xp_h.at[sv.at[j]], rows.at[j], gsem)
                      for j in range(_CHUNK_ROWS)]
                for h in gh:
                    h.wait()
                sh = [pltpu.async_copy(rows.at[j], acc_ref.at[dv.at[j]], ssem,
                                       add=True)
                      for j in range(_CHUNK_ROWS)]
                for h in sh:
                    h.wait()
                return carry

            lax.fori_loop(0, n_steps, step, 0)

        edge_pass(src_h, dst_h, acc, n_base, base_wrows)
        edge_pass(csrc_h, cdst_h, cacc, n_caus, caus_wrows)
        plsc.subcore_barrier()

        sl = pl.ds(s * rows_sub, rows_sub)
        pltpu.sync_copy(acc.at[sl], agg_h.at[c, sl])
        pltpu.sync_copy(cacc.at[sl], cagg_h.at[c, sl])

    return k(xp, srcr, dstr, csrcr, cdstr, zrows)


def _tc_dense(x2d, aggp, caggp, Wm, P, B, T, BT, OD, R, Nb):
    grid = (R // Nb,)

    def body(x_ref, ap_ref, cp_ref, w_ref, p_ref, f_ref, h_ref, l_ref):
        agg = ap_ref[0] + ap_ref[1]
        cagg = cp_ref[0] + cp_ref[1]
        aggT = agg.T
        caggT = cagg.T
        deg = aggT[BT:BT + 1, :]
        cdeg = caggT[BT:BT + 1, :]
        dmask = (deg > 0).astype(jnp.float32)
        cmask = (cdeg > 0).astype(jnp.float32)
        dinv = 1.0 / jnp.maximum(deg, 1.0)
        cinv = 1.0 / jnp.maximum(cdeg, 1.0)

        def mm(Wmat, Xmat):
            return lax.dot_general(
                Wmat, Xmat, (((1,), (0,)), ((), ())),
                precision=lax.Precision.HIGHEST,
                preferred_element_type=jnp.float32)

        def ln(h, gcol, bcol):
            mu = jnp.mean(h, axis=0, keepdims=True)
            xc = h - mu
            var = jnp.mean(xc * xc, axis=0, keepdims=True)
            return gcol * xc * lax.rsqrt(var + 1e-5) + bcol

        pcol = lambda k: p_ref[:, k:k + 1]
        batt = p_ref[0:1, 14:15]

        for b in range(B):
            xb = x_ref[b * T:(b + 1) * T, :]
            maT = aggT[b * T:(b + 1) * T, :] * dinv
            caT = caggT[b * T:(b + 1) * T, :] * cinv

            hp = (mm(w_ref[0], maT) + mm(w_ref[1], xb)
                  + pcol(0) * dmask + pcol(1))
            hn = ln(hp, pcol(6), pcol(7))
            high = jnp.where(hn > 0, hn, 0.1 * hn)

            lp = (mm(w_ref[2], maT) + mm(w_ref[3], caT) + mm(w_ref[4], xb)
                  + pcol(2) * dmask + pcol(3) * cmask + pcol(4))
            lnl = ln(lp, pcol(8), pcol(9))
            low = 0.5 * lnl * (1.0 + jnp.tanh(
                0.7978845608028654 * (lnl + 0.044715 * lnl * lnl * lnl)))

            res = mm(w_ref[5], 2.0 * xb) + pcol(5)
            res = ln(res, pcol(10), pcol(11))

            s = (jnp.sum(high * pcol(12), axis=0, keepdims=True)
                 + jnp.sum(low * pcol(13), axis=0, keepdims=True) + batt)
            alpha = 1.0 / (1.0 + jnp.exp(-s))
            fused = (alpha + 0.3) * high + (1.3 - alpha) * low + 0.1 * res

            f_ref[b] = fused
            h_ref[b] = high
            l_ref[b] = low

    CW = aggp.shape[2]
    out_sd = jax.ShapeDtypeStruct((B, OD, R), jnp.float32)
    return pl.pallas_call(
        body,
        grid=grid,
        in_specs=[
            pl.BlockSpec((BT, Nb), lambda i: (0, i)),
            pl.BlockSpec((_NC, Nb, CW), lambda i: (0, i, 0)),
            pl.BlockSpec((_NC, Nb, CW), lambda i: (0, i, 0)),
            pl.BlockSpec((6, OD, T), lambda i: (0, 0, 0)),
            pl.BlockSpec((OD, 16), lambda i: (0, 0)),
        ],
        out_specs=[
            pl.BlockSpec((B, OD, Nb), lambda i: (0, 0, i)),
            pl.BlockSpec((B, OD, Nb), lambda i: (0, 0, i)),
            pl.BlockSpec((B, OD, Nb), lambda i: (0, 0, i)),
        ],
        out_shape=(out_sd, out_sd, out_sd),
    )(x2d, aggp, caggp, Wm, P)


def kernel(x, edge_index, causal_edge_index, W_high_temp, b_high_temp,
           W_low_temp, b_low_temp, W_hg_nei, W_hg_self, b_hg, W_lg_nei,
           W_lg_self, b_lg, W_lc, W_hr, b_hr, W_lr, b_lr, g_h, be_h, g_l,
           be_l, W_att, b_att, W_gr, b_gr, g_g, be_g):
    B, T, N = x.shape
    E = edge_index.shape[1]
    EC = causal_edge_index.shape[1]
    OD = W_hg_nei.shape[1]
    BT = B * T
    CW = _ceil_to(BT + 1, 16)
    R = _ceil_to(N + 1, _NS * _EDGES_PER_STEP // 16)
    if R % (_NS * 8):
        R = _ceil_to(R, _NS * 8)
    Nb = 1024 if R % 1024 == 0 else 512

    f32 = jnp.float32

    x2d = x.reshape(BT, N)
    x2dp = jnp.pad(x2d, ((0, 0), (0, R - N)))
    xp = jnp.concatenate(
        [x2d.T, jnp.ones((N, 1), f32), jnp.zeros((N, CW - BT - 1), f32)],
        axis=1)

    def pack_edges(ei, ne):
        epad = _ceil_to(ne, _NW * _EDGES_PER_STEP)
        pe = epad - ne
        s = jnp.concatenate([ei[0], jnp.zeros((pe,), jnp.int32)])
        d = jnp.concatenate([ei[1], jnp.full((pe,), N, jnp.int32)])
        return (s.reshape(epad // 128, 128), d.reshape(epad // 128, 128),
                epad // (_NW * _EDGES_PER_STEP))

    srcr, dstr, n_base = pack_edges(edge_index, E)
    csrcr, cdstr, n_caus = pack_edges(causal_edge_index, EC)
    zrows = jnp.zeros((R // _NS, CW), f32)

    Ah = W_high_temp @ W_hg_nei
    Ch = W_high_temp @ (W_hg_self + 0.2 * W_hr)
    bAh = b_high_temp @ W_hg_nei
    ch = b_high_temp @ (W_hg_self + 0.2 * W_hr) + b_hg + 0.2 * b_hr
    Al = W_low_temp @ W_lg_nei
    Alc = W_low_temp @ W_lc
    Cl = W_low_temp @ (W_lg_self + 0.2 * W_lr)
    bAl = b_low_temp @ W_lg_nei
    bAlc = b_low_temp @ W_lc
    cl = b_low_temp @ (W_lg_self + 0.2 * W_lr) + b_lg + 0.2 * b_lr

    Wm = jnp.stack([Ah.T, Ch.T, Al.T, Alc.T, Cl.T, W_gr.T])
    P = jnp.stack(
        [bAh, ch, bAl, bAlc, cl, b_gr, g_h, be_h, g_l, be_l, g_g, be_g,
         W_att[:OD], W_att[OD:], jnp.full((OD,), b_att, f32),
         jnp.zeros((OD,), f32)], axis=1)

    aggp, caggp = _sc_aggregate(xp, srcr, dstr, csrcr, cdstr, zrows,
                                R, CW, n_base, n_caus)
    fused, high, low = _tc_dense(x2dp, aggp, caggp, Wm, P,
                                 B, T, BT, OD, R, Nb)
    return (fused[:, :, :N], high[:, :, :N], low[:, :, :N])


# speedup vs baseline: 43.6214x; 1.1456x over previous
import functools

import jax
import jax.numpy as jnp
from jax import lax
from jax.experimental import pallas as pl
from jax.experimental.pallas import tpu as pltpu
from jax.experimental.pallas import tpu_sc as plsc

_NC = 2
_NS = 16
_NW = _NC * _NS
_CHUNK_ROWS = 4
_EDGES_PER_STEP = _CHUNK_ROWS * 128


def _ceil_to(v, m):
    return (v + m - 1) // m * m


def _sc_aggregate(xp, srcr, dstr, csrcr, cdstr, zrows, R, CW,
                  base_split, caus_split):
    rows_sub = R // _NS

    mesh = plsc.VectorSubcoreMesh(
        core_axis_name="c", subcore_axis_name="s",
        num_cores=_NC, num_subcores=_NS)

    @functools.partial(
        pl.kernel,
        out_type=(jax.ShapeDtypeStruct((_NC, R, CW), jnp.float32),
                  jax.ShapeDtypeStruct((_NC, R, CW), jnp.float32)),
        mesh=mesh,
        compiler_params=pltpu.CompilerParams(use_tc_tiling_on_sc=False),
        scratch_types=(
            pltpu.VMEM_SHARED((R, CW), jnp.float32),
            pltpu.VMEM_SHARED((R, CW), jnp.float32),
            pltpu.VMEM((_CHUNK_ROWS, 128), jnp.int32),
            pltpu.VMEM((_CHUNK_ROWS, 128), jnp.int32),
            pltpu.VMEM((_CHUNK_ROWS, 128, CW), jnp.float32),
            pltpu.SemaphoreType.DMA,
            pltpu.SemaphoreType.DMA,
        ),
    )
    def k(xp_h, src_h, dst_h, csrc_h, cdst_h, z_h, agg_h, cagg_h,
          acc, cacc, sv, dv, rows, gsem, ssem):
        c = lax.axis_index("c")
        s = lax.axis_index("s")

        pltpu.sync_copy(z_h, acc.at[pl.ds(s * rows_sub, rows_sub)])
        pltpu.sync_copy(z_h, cacc.at[pl.ds(s * rows_sub, rows_sub)])
        plsc.subcore_barrier()

        def edge_pass(src_ref, dst_ref, acc_ref, split):
            s0, s1 = split
            n_steps = jnp.where(c == 0, s0, s1)
            base_row = jnp.where(
                c == 0, s * (_CHUNK_ROWS * s0),
                _NS * _CHUNK_ROWS * s0 + s * (_CHUNK_ROWS * s1))

            def step(i, carry):
                r0 = base_row + i * _CHUNK_ROWS
                pltpu.sync_copy(src_ref.at[pl.ds(r0, _CHUNK_ROWS)], sv)
                pltpu.sync_copy(dst_ref.at[pl.ds(r0, _CHUNK_ROWS)], dv)
                gh = [pltpu.async_copy(xp_h.at[sv.at[j]], rows.at[j], gsem)
                      for j in range(_CHUNK_ROWS)]
                for h in gh:
                    h.wait()
                sh = [pltpu.async_copy(rows.at[j], acc_ref.at[dv.at[j]], ssem,
                                       add=True)
                      for j in range(_CHUNK_ROWS)]
                for h in sh:
                    h.wait()
                return carry

            lax.fori_loop(0, n_steps, step, 0)

        edge_pass(src_h, dst_h, acc, base_split)
        edge_pass(csrc_h, cdst_h, cacc, caus_split)
        plsc.subcore_barrier()

        sl = pl.ds(s * rows_sub, rows_sub)
        pltpu.sync_copy(acc.at[sl], agg_h.at[c, sl])
        pltpu.sync_copy(cacc.at[sl], cagg_h.at[c, sl])

    return k(xp, srcr, dstr, csrcr, cdstr, zrows)


def _tc_dense(x2d, aggp, caggp, Wm, P, B, T, BT, OD, R, Nb):
    grid = (R // Nb,)

    def body(x_ref, ap_ref, cp_ref, w_ref, p_ref, f_ref, h_ref, l_ref):
        agg = ap_ref[0] + ap_ref[1]
        cagg = cp_ref[0] + cp_ref[1]
        aggT = agg.T
        caggT = cagg.T
        deg = aggT[BT:BT + 1, :]
        cdeg = caggT[BT:BT + 1, :]
        dmask = (deg > 0).astype(jnp.float32)
        cmask = (cdeg > 0).astype(jnp.float32)
        dinv = 1.0 / jnp.maximum(deg, 1.0)
        cinv = 1.0 / jnp.maximum(cdeg, 1.0)

        def mm(Wmat, Xmat):
            return lax.dot_general(
                Wmat, Xmat, (((1,), (0,)), ((), ())),
                precision=lax.Precision.HIGHEST,
                preferred_element_type=jnp.float32)

        def ln(h, gcol, bcol):
            mu = jnp.mean(h, axis=0, keepdims=True)
            xc = h - mu
            var = jnp.mean(xc * xc, axis=0, keepdims=True)
            return gcol * xc * lax.rsqrt(var + 1e-5) + bcol

        pcol = lambda k: p_ref[:, k:k + 1]
        batt = p_ref[0:1, 14:15]

        for b in range(B):
            xb = x_ref[b * T:(b + 1) * T, :]
            maT = aggT[b * T:(b + 1) * T, :] * dinv
            caT = caggT[b * T:(b + 1) * T, :] * cinv

            hp = (mm(w_ref[0], maT) + mm(w_ref[1], xb)
                  + pcol(0) * dmask + pcol(1))
            hn = ln(hp, pcol(6), pcol(7))
            high = jnp.where(hn > 0, hn, 0.1 * hn)

            lp = (mm(w_ref[2], maT) + mm(w_ref[3], caT) + mm(w_ref[4], xb)
                  + pcol(2) * dmask + pcol(3) * cmask + pcol(4))
            lnl = ln(lp, pcol(8), pcol(9))
            low = 0.5 * lnl * (1.0 + jnp.tanh(
                0.7978845608028654 * (lnl + 0.044715 * lnl * lnl * lnl)))

            res = mm(w_ref[5], 2.0 * xb) + pcol(5)
            res = ln(res, pcol(10), pcol(11))

            s = (jnp.sum(high * pcol(12), axis=0, keepdims=True)
                 + jnp.sum(low * pcol(13), axis=0, keepdims=True) + batt)
            alpha = 1.0 / (1.0 + jnp.exp(-s))
            fused = (alpha + 0.3) * high + (1.3 - alpha) * low + 0.1 * res

            f_ref[b] = fused
            h_ref[b] = high
            l_ref[b] = low

    CW = aggp.shape[2]
    out_sd = jax.ShapeDtypeStruct((B, OD, R), jnp.float32)
    return pl.pallas_call(
        body,
        grid=grid,
        in_specs=[
            pl.BlockSpec((BT, Nb), lambda i: (0, i)),
            pl.BlockSpec((_NC, Nb, CW), lambda i: (0, i, 0)),
            pl.BlockSpec((_NC, Nb, CW), lambda i: (0, i, 0)),
            pl.BlockSpec((6, OD, T), lambda i: (0, 0, 0)),
            pl.BlockSpec((OD, 16), lambda i: (0, 0)),
        ],
        out_specs=[
            pl.BlockSpec((B, OD, Nb), lambda i: (0, 0, i)),
            pl.BlockSpec((B, OD, Nb), lambda i: (0, 0, i)),
            pl.BlockSpec((B, OD, Nb), lambda i: (0, 0, i)),
        ],
        out_shape=(out_sd, out_sd, out_sd),
    )(x2d, aggp, caggp, Wm, P)


def kernel(x, edge_index, causal_edge_index, W_high_temp, b_high_temp,
           W_low_temp, b_low_temp, W_hg_nei, W_hg_self, b_hg, W_lg_nei,
           W_lg_self, b_lg, W_lc, W_hr, b_hr, W_lr, b_lr, g_h, be_h, g_l,
           be_l, W_att, b_att, W_gr, b_gr, g_g, be_g):
    B, T, N = x.shape
    E = edge_index.shape[1]
    EC = causal_edge_index.shape[1]
    OD = W_hg_nei.shape[1]
    BT = B * T
    CW = _ceil_to(BT + 1, 16)
    R = _ceil_to(N + 1, _NS * _EDGES_PER_STEP // 16)
    if R % (_NS * 8):
        R = _ceil_to(R, _NS * 8)
    Nb = 1024 if R % 1024 == 0 else 512

    f32 = jnp.float32

    x2d = x.reshape(BT, N)
    x2dp = jnp.pad(x2d, ((0, 0), (0, R - N)))
    xp = jnp.concatenate(
        [x2d.T, jnp.ones((N, 1), f32), jnp.zeros((N, CW - BT - 1), f32)],
        axis=1)

    _F0 = 0.73

    def pack_edges(ei, ne):
        epad = _ceil_to(ne, _NS * _EDGES_PER_STEP)
        pe = epad - ne
        s = jnp.concatenate([ei[0], jnp.zeros((pe,), jnp.int32)])
        d = jnp.concatenate([ei[1], jnp.full((pe,), N, jnp.int32)])
        tot = epad // (_NS * _EDGES_PER_STEP)
        s0 = min(max(int(round(_F0 * tot)), 0), tot)
        return (s.reshape(epad // 128, 128), d.reshape(epad // 128, 128),
                (s0, tot - s0))

    srcr, dstr, base_split = pack_edges(edge_index, E)
    csrcr, cdstr, caus_split = pack_edges(causal_edge_index, EC)
    zrows = jnp.zeros((R // _NS, CW), f32)

    Ah = W_high_temp @ W_hg_nei
    Ch = W_high_temp @ (W_hg_self + 0.2 * W_hr)
    bAh = b_high_temp @ W_hg_nei
    ch = b_high_temp @ (W_hg_self + 0.2 * W_hr) + b_hg + 0.2 * b_hr
    Al = W_low_temp @ W_lg_nei
    Alc = W_low_temp @ W_lc
    Cl = W_low_temp @ (W_lg_self + 0.2 * W_lr)
    bAl = b_low_temp @ W_lg_nei
    bAlc = b_low_temp @ W_lc
    cl = b_low_temp @ (W_lg_self + 0.2 * W_lr) + b_lg + 0.2 * b_lr

    Wm = jnp.stack([Ah.T, Ch.T, Al.T, Alc.T, Cl.T, W_gr.T])
    P = jnp.stack(
        [bAh, ch, bAl, bAlc, cl, b_gr, g_h, be_h, g_l, be_l, g_g, be_g,
         W_att[:OD], W_att[OD:], jnp.full((OD,), b_att, f32),
         jnp.zeros((OD,), f32)], axis=1)

    aggp, caggp = _sc_aggregate(xp, srcr, dstr, csrcr, cdstr, zrows,
                                R, CW, base_split, caus_split)
    fused, high, low = _tc_dense(x2dp, aggp, caggp, Wm, P,
                                 B, T, BT, OD, R, Nb)
    return (fused[:, :, :N], high[:, :, :N], low[:, :, :N])


# speedup vs baseline: 46.9011x; 1.0752x over previous
import functools

import jax
import jax.numpy as jnp
from jax import lax
from jax.experimental import pallas as pl
from jax.experimental.pallas import tpu as pltpu
from jax.experimental.pallas import tpu_sc as plsc

_NC = 2
_NS = 16
_NW = _NC * _NS
_CHUNK_ROWS = 4
_EDGES_PER_STEP = _CHUNK_ROWS * 128


def _ceil_to(v, m):
    return (v + m - 1) // m * m


def _sc_aggregate(xp, srcr, dstr, csrcr, cdstr, R, CW,
                  base_split, caus_split):
    rows_sub = R // _NS

    mesh = plsc.VectorSubcoreMesh(
        core_axis_name="c", subcore_axis_name="s",
        num_cores=_NC, num_subcores=_NS)

    @functools.partial(
        pl.kernel,
        out_type=(jax.ShapeDtypeStruct((_NC, R, CW), jnp.float32),
                  jax.ShapeDtypeStruct((_NC, R, CW), jnp.float32)),
        mesh=mesh,
        compiler_params=pltpu.CompilerParams(use_tc_tiling_on_sc=False),
        scratch_types=(
            pltpu.VMEM_SHARED((R, CW), jnp.float32),
            pltpu.VMEM_SHARED((R, CW), jnp.float32),
            pltpu.VMEM((_CHUNK_ROWS, 128), jnp.int32),
            pltpu.VMEM((_CHUNK_ROWS, 128), jnp.int32),
            pltpu.VMEM((_CHUNK_ROWS, 128, CW), jnp.float32),
            pltpu.VMEM((128, CW), jnp.float32),
            pltpu.SemaphoreType.DMA,
            pltpu.SemaphoreType.DMA,
        ),
    )
    def k(xp_h, src_h, dst_h, csrc_h, cdst_h, agg_h, cagg_h,
          acc, cacc, sv, dv, rows, zbuf, gsem, ssem):
        c = lax.axis_index("c")
        s = lax.axis_index("s")

        z16 = jnp.zeros((16,), jnp.float32)

        def zrow(i, carry):
            for kk in range(CW // 16):
                zbuf[i, pl.ds(kk * 16, 16)] = z16
            return carry

        lax.fori_loop(0, 128, zrow, 0)
        for t in range(rows_sub // 128):
            off = s * rows_sub + t * 128
            pltpu.sync_copy(zbuf, acc.at[pl.ds(off, 128)])
            pltpu.sync_copy(zbuf, cacc.at[pl.ds(off, 128)])
        plsc.subcore_barrier()

        def edge_pass(src_ref, dst_ref, acc_ref, split):
            s0, s1 = split
            n_steps = jnp.where(c == 0, s0, s1)
            base_row = jnp.where(
                c == 0, s * (_CHUNK_ROWS * s0),
                _NS * _CHUNK_ROWS * s0 + s * (_CHUNK_ROWS * s1))

            def step(i, carry):
                r0 = base_row + i * _CHUNK_ROWS
                pltpu.sync_copy(src_ref.at[pl.ds(r0, _CHUNK_ROWS)], sv)
                pltpu.sync_copy(dst_ref.at[pl.ds(r0, _CHUNK_ROWS)], dv)
                gh = [pltpu.async_copy(xp_h.at[sv.at[j]], rows.at[j], gsem)
                      for j in range(_CHUNK_ROWS)]
                for h in gh:
                    h.wait()
                sh = [pltpu.async_copy(rows.at[j], acc_ref.at[dv.at[j]], ssem,
                                       add=True)
                      for j in range(_CHUNK_ROWS)]
                for h in sh:
                    h.wait()
                return carry

            lax.fori_loop(0, n_steps, step, 0)

        edge_pass(src_h, dst_h, acc, base_split)
        edge_pass(csrc_h, cdst_h, cacc, caus_split)
        plsc.subcore_barrier()

        sl = pl.ds(s * rows_sub, rows_sub)
        pltpu.sync_copy(acc.at[sl], agg_h.at[c, sl])
        pltpu.sync_copy(cacc.at[sl], cagg_h.at[c, sl])

    return k(xp, srcr, dstr, csrcr, cdstr)


def _tc_dense(x2d, aggp, caggp, Wm, P, B, T, BT, OD, R, Nb):
    grid = (R // Nb,)

    def body(x_ref, ap_ref, cp_ref, w_ref, p_ref, f_ref, h_ref, l_ref):
        agg = ap_ref[0] + ap_ref[1]
        cagg = cp_ref[0] + cp_ref[1]
        aggT = agg.T
        caggT = cagg.T
        deg = aggT[BT:BT + 1, :]
        cdeg = caggT[BT:BT + 1, :]
        dmask = (deg > 0).astype(jnp.float32)
        cmask = (cdeg > 0).astype(jnp.float32)
        dinv = 1.0 / jnp.maximum(deg, 1.0)
        cinv = 1.0 / jnp.maximum(cdeg, 1.0)

        def mm(Wmat, Xmat):
            return lax.dot_general(
                Wmat, Xmat, (((1,), (0,)), ((), ())),
                precision=lax.Precision.HIGHEST,
                preferred_element_type=jnp.float32)

        def ln(h, gcol, bcol):
            mu = jnp.mean(h, axis=0, keepdims=True)
            xc = h - mu
            var = jnp.mean(xc * xc, axis=0, keepdims=True)
            return gcol * xc * lax.rsqrt(var + 1e-5) + bcol

        pcol = lambda k: p_ref[:, k:k + 1]
        batt = p_ref[0:1, 14:15]

        for b in range(B):
            xb = x_ref[b * T:(b + 1) * T, :]
            maT = aggT[b * T:(b + 1) * T, :] * dinv
            caT = caggT[b * T:(b + 1) * T, :] * cinv

            hp = (mm(w_ref[0], maT) + mm(w_ref[1], xb)
                  + pcol(0) * dmask + pcol(1))
            hn = ln(hp, pcol(6), pcol(7))
            high = jnp.where(hn > 0, hn, 0.1 * hn)

            lp = (mm(w_ref[2], maT) + mm(w_ref[3], caT) + mm(w_ref[4], xb)
                  + pcol(2) * dmask + pcol(3) * cmask + pcol(4))
            lnl = ln(lp, pcol(8), pcol(9))
            low = 0.5 * lnl * (1.0 + jnp.tanh(
                0.7978845608028654 * (lnl + 0.044715 * lnl * lnl * lnl)))

            res = mm(w_ref[5], 2.0 * xb) + pcol(5)
            res = ln(res, pcol(10), pcol(11))

            s = (jnp.sum(high * pcol(12), axis=0, keepdims=True)
                 + jnp.sum(low * pcol(13), axis=0, keepdims=True) + batt)
            alpha = 1.0 / (1.0 + jnp.exp(-s))
            fused = (alpha + 0.3) * high + (1.3 - alpha) * low + 0.1 * res

            f_ref[b] = fused
            h_ref[b] = high
            l_ref[b] = low

    CW = aggp.shape[2]
    out_sd = jax.ShapeDtypeStruct((B, OD, R), jnp.float32)
    return pl.pallas_call(
        body,
        grid=grid,
        in_specs=[
            pl.BlockSpec((BT, Nb), lambda i: (0, i)),
            pl.BlockSpec((_NC, Nb, CW), lambda i: (0, i, 0)),
            pl.BlockSpec((_NC, Nb, CW), lambda i: (0, i, 0)),
            pl.BlockSpec((6, OD, T), lambda i: (0, 0, 0)),
            pl.BlockSpec((OD, 16), lambda i: (0, 0)),
        ],
        out_specs=[
            pl.BlockSpec((B, OD, Nb), lambda i: (0, 0, i)),
            pl.BlockSpec((B, OD, Nb), lambda i: (0, 0, i)),
            pl.BlockSpec((B, OD, Nb), lambda i: (0, 0, i)),
        ],
        out_shape=(out_sd, out_sd, out_sd),
    )(x2d, aggp, caggp, Wm, P)


def kernel(x, edge_index, causal_edge_index, W_high_temp, b_high_temp,
           W_low_temp, b_low_temp, W_hg_nei, W_hg_self, b_hg, W_lg_nei,
           W_lg_self, b_lg, W_lc, W_hr, b_hr, W_lr, b_lr, g_h, be_h, g_l,
           be_l, W_att, b_att, W_gr, b_gr, g_g, be_g):
    B, T, N = x.shape
    E = edge_index.shape[1]
    EC = causal_edge_index.shape[1]
    OD = W_hg_nei.shape[1]
    BT = B * T
    CW = _ceil_to(BT + 1, 16)
    R = _ceil_to(N + 1, _NS * _EDGES_PER_STEP // 16)
    if R % (_NS * 8):
        R = _ceil_to(R, _NS * 8)
    Nb = 1024 if R % 1024 == 0 else 512

    f32 = jnp.float32

    x2d = x.reshape(BT, N)
    x2dp = jnp.pad(x2d, ((0, 0), (0, R - N)))
    xp = jnp.concatenate(
        [x2d.T, jnp.ones((N, 1), f32), jnp.zeros((N, CW - BT - 1), f32)],
        axis=1)

    _F0 = 0.85

    def pack_edges(ei, ne):
        epad = _ceil_to(ne, _NS * _EDGES_PER_STEP)
        pe = epad - ne
        s = jnp.concatenate([ei[0], jnp.zeros((pe,), jnp.int32)])
        d = jnp.concatenate([ei[1], jnp.full((pe,), N, jnp.int32)])
        tot = epad // (_NS * _EDGES_PER_STEP)
        s0 = min(max(int(round(_F0 * tot)), 0), tot)
        return (s.reshape(epad // 128, 128), d.reshape(epad // 128, 128),
                (s0, tot - s0))

    srcr, dstr, base_split = pack_edges(edge_index, E)
    csrcr, cdstr, caus_split = pack_edges(causal_edge_index, EC)

    Ah = W_high_temp @ W_hg_nei
    Ch = W_high_temp @ (W_hg_self + 0.2 * W_hr)
    bAh = b_high_temp @ W_hg_nei
    ch = b_high_temp @ (W_hg_self + 0.2 * W_hr) + b_hg + 0.2 * b_hr
    Al = W_low_temp @ W_lg_nei
    Alc = W_low_temp @ W_lc
    Cl = W_low_temp @ (W_lg_self + 0.2 * W_lr)
    bAl = b_low_temp @ W_lg_nei
    bAlc = b_low_temp @ W_lc
    cl = b_low_temp @ (W_lg_self + 0.2 * W_lr) + b_lg + 0.2 * b_lr

    Wm = jnp.stack([Ah.T, Ch.T, Al.T, Alc.T, Cl.T, W_gr.T])
    P = jnp.stack(
        [bAh, ch, bAl, bAlc, cl, b_gr, g_h, be_h, g_l, be_l, g_g, be_g,
         W_att[:OD], W_att[OD:], jnp.full((OD,), b_att, f32),
         jnp.zeros((OD,), f32)], axis=1)

    aggp, caggp = _sc_aggregate(xp, srcr, dstr, csrcr, cdstr,
                                R, CW, base_split, caus_split)
    fused, high, low = _tc_dense(x2dp, aggp, caggp, Wm, P,
                                 B, T, BT, OD, R, Nb)
    return (fused[:, :, :N], high[:, :, :N], low[:, :, :N])


# speedup vs baseline: 48.6114x; 1.0365x over previous
import functools

import jax
import jax.numpy as jnp
from jax import lax
from jax.experimental import pallas as pl
from jax.experimental.pallas import tpu as pltpu
from jax.experimental.pallas import tpu_sc as plsc

_NC = 2
_NS = 16
_NW = _NC * _NS
_CHUNK_ROWS = 4
_EDGES_PER_STEP = _CHUNK_ROWS * 128


def _ceil_to(v, m):
    return (v + m - 1) // m * m


def _sc_aggregate(xp, srcr, dstr, csrcr, cdstr, R, CW,
                  base_split, caus_split):
    rows_sub = R // _NS

    mesh = plsc.VectorSubcoreMesh(
        core_axis_name="c", subcore_axis_name="s",
        num_cores=_NC, num_subcores=_NS)

    @functools.partial(
        pl.kernel,
        out_type=(jax.ShapeDtypeStruct((_NC, R, CW), jnp.float32),
                  jax.ShapeDtypeStruct((_NC, R, CW), jnp.float32)),
        mesh=mesh,
        compiler_params=pltpu.CompilerParams(use_tc_tiling_on_sc=False),
        scratch_types=(
            pltpu.VMEM_SHARED((R, CW), jnp.float32),
            pltpu.VMEM_SHARED((R, CW), jnp.float32),
            pltpu.VMEM((_CHUNK_ROWS, 128), jnp.int32),
            pltpu.VMEM((_CHUNK_ROWS, 128), jnp.int32),
            pltpu.VMEM((_CHUNK_ROWS, 128, CW), jnp.float32),
            pltpu.VMEM((128, CW), jnp.float32),
            pltpu.SemaphoreType.DMA,
            pltpu.SemaphoreType.DMA,
        ),
    )
    def k(xp_h, src_h, dst_h, csrc_h, cdst_h, agg_h, cagg_h,
          acc, cacc, sv, dv, rows, zbuf, gsem, ssem):
        c = lax.axis_index("c")
        s = lax.axis_index("s")

        z16 = jnp.zeros((16,), jnp.float32)

        def zrow(i, carry):
            for kk in range(CW // 16):
                zbuf[i, pl.ds(kk * 16, 16)] = z16
            return carry

        lax.fori_loop(0, 128, zrow, 0)
        for t in range(rows_sub // 128):
            off = s * rows_sub + t * 128
            pltpu.sync_copy(zbuf, acc.at[pl.ds(off, 128)])
            pltpu.sync_copy(zbuf, cacc.at[pl.ds(off, 128)])
        plsc.subcore_barrier()

        def edge_pass(src_ref, dst_ref, acc_ref, split):
            s0, s1 = split
            n_steps = jnp.where(c == 0, s0, s1)
            base_row = jnp.where(
                c == 0, s * (_CHUNK_ROWS * s0),
                _NS * _CHUNK_ROWS * s0 + s * (_CHUNK_ROWS * s1))

            def step(i, carry):
                r0 = base_row + i * _CHUNK_ROWS
                pltpu.sync_copy(src_ref.at[pl.ds(r0, _CHUNK_ROWS)], sv)
                pltpu.sync_copy(dst_ref.at[pl.ds(r0, _CHUNK_ROWS)], dv)
                gh = [pltpu.async_copy(xp_h.at[sv.at[j]], rows.at[j], gsem)
                      for j in range(_CHUNK_ROWS)]
                for h in gh:
                    h.wait()
                sh = [pltpu.async_copy(rows.at[j], acc_ref.at[dv.at[j]], ssem,
                                       add=True)
                      for j in range(_CHUNK_ROWS)]
                for h in sh:
                    h.wait()
                return carry

            lax.fori_loop(0, n_steps, step, 0)

        edge_pass(src_h, dst_h, acc, base_split)
        edge_pass(csrc_h, cdst_h, cacc, caus_split)
        plsc.subcore_barrier()

        sl = pl.ds(s * rows_sub, rows_sub)
        pltpu.sync_copy(acc.at[sl], agg_h.at[c, sl])
        pltpu.sync_copy(cacc.at[sl], cagg_h.at[c, sl])

    return k(xp, srcr, dstr, csrcr, cdstr)


def _tc_dense(x2d, aggp, caggp, Wm, P, B, T, BT, OD, N, R, Nb):
    grid = (pl.cdiv(N, Nb),)

    def body(x_ref, ap_ref, cp_ref, w_ref, p_ref, f_ref, h_ref, l_ref):
        agg = ap_ref[0] + ap_ref[1]
        cagg = cp_ref[0] + cp_ref[1]
        aggT = agg.T
        caggT = cagg.T
        deg = aggT[BT:BT + 1, :]
        cdeg = caggT[BT:BT + 1, :]
        dmask = (deg > 0).astype(jnp.float32)
        cmask = (cdeg > 0).astype(jnp.float32)
        dinv = 1.0 / jnp.maximum(deg, 1.0)
        cinv = 1.0 / jnp.maximum(cdeg, 1.0)

        def mm(Wmat, Xmat):
            return lax.dot_general(
                Wmat, Xmat, (((1,), (0,)), ((), ())),
                precision=lax.Precision.HIGHEST,
                preferred_element_type=jnp.float32)

        def ln(h, gcol, bcol):
            mu = jnp.mean(h, axis=0, keepdims=True)
            xc = h - mu
            var = jnp.mean(xc * xc, axis=0, keepdims=True)
            return gcol * xc * lax.rsqrt(var + 1e-5) + bcol

        pcol = lambda k: p_ref[:, k:k + 1]
        batt = p_ref[0:1, 14:15]

        for b in range(B):
            xb = x_ref[b * T:(b + 1) * T, :]
            maT = aggT[b * T:(b + 1) * T, :] * dinv
            caT = caggT[b * T:(b + 1) * T, :] * cinv

            hp = (mm(w_ref[0], maT) + mm(w_ref[1], xb)
                  + pcol(0) * dmask + pcol(1))
            hn = ln(hp, pcol(6), pcol(7))
            high = jnp.where(hn > 0, hn, 0.1 * hn)

            lp = (mm(w_ref[2], maT) + mm(w_ref[3], caT) + mm(w_ref[4], xb)
                  + pcol(2) * dmask + pcol(3) * cmask + pcol(4))
            lnl = ln(lp, pcol(8), pcol(9))
            low = 0.5 * lnl * (1.0 + jnp.tanh(
                0.7978845608028654 * (lnl + 0.044715 * lnl * lnl * lnl)))

            res = mm(w_ref[5], 2.0 * xb) + pcol(5)
            res = ln(res, pcol(10), pcol(11))

            s = (jnp.sum(high * pcol(12), axis=0, keepdims=True)
                 + jnp.sum(low * pcol(13), axis=0, keepdims=True) + batt)
            alpha = 1.0 / (1.0 + jnp.exp(-s))
            fused = (alpha + 0.3) * high + (1.3 - alpha) * low + 0.1 * res

            f_ref[b] = fused
            h_ref[b] = high
            l_ref[b] = low

    CW = aggp.shape[2]
    out_sd = jax.ShapeDtypeStruct((B, OD, N), jnp.float32)
    return pl.pallas_call(
        body,
        grid=grid,
        in_specs=[
            pl.BlockSpec((BT, Nb), lambda i: (0, i)),
            pl.BlockSpec((_NC, Nb, CW), lambda i: (0, i, 0)),
            pl.BlockSpec((_NC, Nb, CW), lambda i: (0, i, 0)),
            pl.BlockSpec((6, OD, T), lambda i: (0, 0, 0)),
            pl.BlockSpec((OD, 16), lambda i: (0, 0)),
        ],
        out_specs=[
            pl.BlockSpec((B, OD, Nb), lambda i: (0, 0, i)),
            pl.BlockSpec((B, OD, Nb), lambda i: (0, 0, i)),
            pl.BlockSpec((B, OD, Nb), lambda i: (0, 0, i)),
        ],
        out_shape=(out_sd, out_sd, out_sd),
    )(x2d, aggp, caggp, Wm, P)


def kernel(x, edge_index, causal_edge_index, W_high_temp, b_high_temp,
           W_low_temp, b_low_temp, W_hg_nei, W_hg_self, b_hg, W_lg_nei,
           W_lg_self, b_lg, W_lc, W_hr, b_hr, W_lr, b_lr, g_h, be_h, g_l,
           be_l, W_att, b_att, W_gr, b_gr, g_g, be_g):
    B, T, N = x.shape
    E = edge_index.shape[1]
    EC = causal_edge_index.shape[1]
    OD = W_hg_nei.shape[1]
    BT = B * T
    CW = _ceil_to(BT + 1, 16)
    R = _ceil_to(N + 1, _NS * _EDGES_PER_STEP // 16)
    if R % (_NS * 8):
        R = _ceil_to(R, _NS * 8)
    Nb = 1024 if R % 1024 == 0 else 512

    f32 = jnp.float32

    x2d = x.reshape(BT, N)
    xp = jnp.concatenate(
        [x2d.T, jnp.ones((N, 1), f32), jnp.zeros((N, CW - BT - 1), f32)],
        axis=1)

    _F0 = 0.85

    def pack_edges(ei, ne):
        epad = _ceil_to(ne, _NS * _EDGES_PER_STEP)
        pe = epad - ne
        s = jnp.concatenate([ei[0], jnp.zeros((pe,), jnp.int32)])
        d = jnp.concatenate([ei[1], jnp.full((pe,), N, jnp.int32)])
        tot = epad // (_NS * _EDGES_PER_STEP)
        s0 = min(max(int(round(_F0 * tot)), 0), tot)
        return (s.reshape(epad // 128, 128), d.reshape(epad // 128, 128),
                (s0, tot - s0))

    srcr, dstr, base_split = pack_edges(edge_index, E)
    csrcr, cdstr, caus_split = pack_edges(causal_edge_index, EC)

    Ah = W_high_temp @ W_hg_nei
    Ch = W_high_temp @ (W_hg_self + 0.2 * W_hr)
    bAh = b_high_temp @ W_hg_nei
    ch = b_high_temp @ (W_hg_self + 0.2 * W_hr) + b_hg + 0.2 * b_hr
    Al = W_low_temp @ W_lg_nei
    Alc = W_low_temp @ W_lc
    Cl = W_low_temp @ (W_lg_self + 0.2 * W_lr)
    bAl = b_low_temp @ W_lg_nei
    bAlc = b_low_temp @ W_lc
    cl = b_low_temp @ (W_lg_self + 0.2 * W_lr) + b_lg + 0.2 * b_lr

    Wm = jnp.stack([Ah.T, Ch.T, Al.T, Alc.T, Cl.T, W_gr.T])
    P = jnp.stack(
        [bAh, ch, bAl, bAlc, cl, b_gr, g_h, be_h, g_l, be_l, g_g, be_g,
         W_att[:OD], W_att[OD:], jnp.full((OD,), b_att, f32),
         jnp.zeros((OD,), f32)], axis=1)

    aggp, caggp = _sc_aggregate(xp, srcr, dstr, csrcr, cdstr,
                                R, CW, base_split, caus_split)
    fused, high, low = _tc_dense(x2d, aggp, caggp, Wm, P,
                                 B, T, BT, OD, N, R, Nb)
    return (fused, high, low)


# speedup vs baseline: 52.6542x; 1.0832x over previous
import functools

import jax
import jax.numpy as jnp
from jax import lax
from jax.experimental import pallas as pl
from jax.experimental.pallas import tpu as pltpu
from jax.experimental.pallas import tpu_sc as plsc

_NC = 2
_NS = 16
_NW = _NC * _NS
_CHUNK_ROWS = 4
_EDGES_PER_STEP = _CHUNK_ROWS * 128


def _ceil_to(v, m):
    return (v + m - 1) // m * m


def _sc_aggregate(xp, srcr, dstr, csrcr, cdstr, R, CW,
                  base_split, caus_split):
    rows_sub = R // _NS

    mesh = plsc.VectorSubcoreMesh(
        core_axis_name="c", subcore_axis_name="s",
        num_cores=_NC, num_subcores=_NS)

    @functools.partial(
        pl.kernel,
        out_type=(jax.ShapeDtypeStruct((_NC, R, CW), jnp.float32),
                  jax.ShapeDtypeStruct((_NC, R, CW), jnp.float32)),
        mesh=mesh,
        compiler_params=pltpu.CompilerParams(use_tc_tiling_on_sc=False),
        scratch_types=(
            pltpu.VMEM_SHARED((R, CW), jnp.float32),
            pltpu.VMEM_SHARED((R, CW), jnp.float32),
            pltpu.VMEM((_CHUNK_ROWS, 128), jnp.int32),
            pltpu.VMEM((_CHUNK_ROWS, 128), jnp.int32),
            pltpu.VMEM((_CHUNK_ROWS, 128, CW), jnp.float32),
            pltpu.VMEM((128, CW), jnp.float32),
            pltpu.SemaphoreType.DMA,
            pltpu.SemaphoreType.DMA,
        ),
    )
    def k(xp_h, src_h, dst_h, csrc_h, cdst_h, agg_h, cagg_h,
          acc, cacc, sv, dv, rows, zbuf, gsem, ssem):
        c = lax.axis_index("c")
        s = lax.axis_index("s")

        z16 = jnp.zeros((16,), jnp.float32)

        def zrow(i, carry):
            for kk in range(CW // 16):
                zbuf[i, pl.ds(kk * 16, 16)] = z16
            return carry

        lax.fori_loop(0, 128, zrow, 0)
        for t in range(rows_sub // 128):
            off = s * rows_sub + t * 128
            pltpu.sync_copy(zbuf, acc.at[pl.ds(off, 128)])
            pltpu.sync_copy(zbuf, cacc.at[pl.ds(off, 128)])
        plsc.subcore_barrier()

        def edge_pass(src_ref, dst_ref, acc_ref, split):
            s0, s1 = split
            n_steps = jnp.where(c == 0, s0, s1)
            base_row = jnp.where(
                c == 0, s * (_CHUNK_ROWS * s0),
                _NS * _CHUNK_ROWS * s0 + s * (_CHUNK_ROWS * s1))

            def step(i, carry):
                r0 = base_row + i * _CHUNK_ROWS
                pltpu.sync_copy(src_ref.at[pl.ds(r0, _CHUNK_ROWS)], sv)
                pltpu.sync_copy(dst_ref.at[pl.ds(r0, _CHUNK_ROWS)], dv)
                gh = [pltpu.async_copy(xp_h.at[sv.at[j]], rows.at[j], gsem)
                      for j in range(_CHUNK_ROWS)]
                for h in gh:
                    h.wait()
                sh = [pltpu.async_copy(rows.at[j], acc_ref.at[dv.at[j]], ssem,
                                       add=True)
                      for j in range(_CHUNK_ROWS)]
                for h in sh:
                    h.wait()
                return carry

            lax.fori_loop(0, n_steps, step, 0)

        edge_pass(src_h, dst_h, acc, base_split)
        edge_pass(csrc_h, cdst_h, cacc, caus_split)
        plsc.subcore_barrier()

        sl = pl.ds(s * rows_sub, rows_sub)
        pltpu.sync_copy(acc.at[sl], agg_h.at[c, sl])
        pltpu.sync_copy(cacc.at[sl], cagg_h.at[c, sl])

    return k(xp, srcr, dstr, csrcr, cdstr)


def _tc_dense(x2d, aggp, caggp, Wm, P, B, T, BT, OD, N, R, Nb):
    grid = (pl.cdiv(N, Nb),)

    def body(x_ref, ap_ref, cp_ref, w_ref, p_ref, f_ref, h_ref, l_ref):
        agg = ap_ref[0] + ap_ref[1]
        cagg = cp_ref[0] + cp_ref[1]
        aggT = agg.T
        caggT = cagg.T
        deg = aggT[BT:BT + 1, :]
        cdeg = caggT[BT:BT + 1, :]
        dmask = (deg > 0).astype(jnp.float32)
        cmask = (cdeg > 0).astype(jnp.float32)
        dinv = 1.0 / jnp.maximum(deg, 1.0)
        cinv = 1.0 / jnp.maximum(cdeg, 1.0)

        def ln(h, gcol, bcol):
            mu = jnp.mean(h, axis=0, keepdims=True)
            xc = h - mu
            var = jnp.mean(xc * xc, axis=0, keepdims=True)
            return gcol * xc * lax.rsqrt(var + 1e-5) + bcol

        pcol = lambda k: p_ref[:, k:k + 1]
        batt = p_ref[0:1, 14:15]

        for b in range(B):
            xb = x_ref[b * T:(b + 1) * T, :]
            maT = aggT[b * T:(b + 1) * T, :] * dinv
            caT = caggT[b * T:(b + 1) * T, :] * cinv

            F = jnp.concatenate([maT, caT, xb], axis=0)
            out = lax.dot_general(
                w_ref[...], F, (((1,), (0,)), ((), ())),
                precision=lax.Precision.HIGHEST,
                preferred_element_type=jnp.float32)

            hp = out[0:OD] + pcol(0) * dmask + pcol(1)
            hn = ln(hp, pcol(6), pcol(7))
            high = jnp.where(hn > 0, hn, 0.1 * hn)

            lp = (out[OD:2 * OD]
                  + pcol(2) * dmask + pcol(3) * cmask + pcol(4))
            lnl = ln(lp, pcol(8), pcol(9))
            low = 0.5 * lnl * (1.0 + jnp.tanh(
                0.7978845608028654 * (lnl + 0.044715 * lnl * lnl * lnl)))

            res = out[2 * OD:3 * OD] + pcol(5)
            res = ln(res, pcol(10), pcol(11))

            s = (jnp.sum(high * pcol(12), axis=0, keepdims=True)
                 + jnp.sum(low * pcol(13), axis=0, keepdims=True) + batt)
            alpha = 1.0 / (1.0 + jnp.exp(-s))
            fused = (alpha + 0.3) * high + (1.3 - alpha) * low + 0.1 * res

            f_ref[b] = fused
            h_ref[b] = high
            l_ref[b] = low

    CW = aggp.shape[2]
    out_sd = jax.ShapeDtypeStruct((B, OD, N), jnp.float32)
    return pl.pallas_call(
        body,
        grid=grid,
        in_specs=[
            pl.BlockSpec((BT, Nb), lambda i: (0, i)),
            pl.BlockSpec((_NC, Nb, CW), lambda i: (0, i, 0)),
            pl.BlockSpec((_NC, Nb, CW), lambda i: (0, i, 0)),
            pl.BlockSpec((3 * OD, 3 * T), lambda i: (0, 0)),
            pl.BlockSpec((OD, 16), lambda i: (0, 0)),
        ],
        out_specs=[
            pl.BlockSpec((B, OD, Nb), lambda i: (0, 0, i)),
            pl.BlockSpec((B, OD, Nb), lambda i: (0, 0, i)),
            pl.BlockSpec((B, OD, Nb), lambda i: (0, 0, i)),
        ],
        out_shape=(out_sd, out_sd, out_sd),
    )(x2d, aggp, caggp, Wm, P)


def kernel(x, edge_index, causal_edge_index, W_high_temp, b_high_temp,
           W_low_temp, b_low_temp, W_hg_nei, W_hg_self, b_hg, W_lg_nei,
           W_lg_self, b_lg, W_lc, W_hr, b_hr, W_lr, b_lr, g_h, be_h, g_l,
           be_l, W_att, b_att, W_gr, b_gr, g_g, be_g):
    B, T, N = x.shape
    E = edge_index.shape[1]
    EC = causal_edge_index.shape[1]
    OD = W_hg_nei.shape[1]
    BT = B * T
    CW = _ceil_to(BT + 1, 16)
    R = _ceil_to(N + 1, _NS * _EDGES_PER_STEP // 16)
    if R % (_NS * 8):
        R = _ceil_to(R, _NS * 8)
    Nb = 1024 if R % 1024 == 0 else 512

    f32 = jnp.float32

    x2d = x.reshape(BT, N)
    xp = jnp.concatenate(
        [x2d.T, jnp.ones((N, 1), f32), jnp.zeros((N, CW - BT - 1), f32)],
        axis=1)

    _F0 = 0.85

    def pack_edges(ei, ne):
        epad = _ceil_to(ne, _NS * _EDGES_PER_STEP)
        pe = epad - ne
        s = jnp.concatenate([ei[0], jnp.zeros((pe,), jnp.int32)])
        d = jnp.concatenate([ei[1], jnp.full((pe,), N, jnp.int32)])
        tot = epad // (_NS * _EDGES_PER_STEP)
        s0 = min(max(int(round(_F0 * tot)), 0), tot)
        return (s.reshape(epad // 128, 128), d.reshape(epad // 128, 128),
                (s0, tot - s0))

    srcr, dstr, base_split = pack_edges(edge_index, E)
    csrcr, cdstr, caus_split = pack_edges(causal_edge_index, EC)

    Ah = W_high_temp @ W_hg_nei
    Ch = W_high_temp @ (W_hg_self + 0.2 * W_hr)
    bAh = b_high_temp @ W_hg_nei
    ch = b_high_temp @ (W_hg_self + 0.2 * W_hr) + b_hg + 0.2 * b_hr
    Al = W_low_temp @ W_lg_nei
    Alc = W_low_temp @ W_lc
    Cl = W_low_temp @ (W_lg_self + 0.2 * W_lr)
    bAl = b_low_temp @ W_lg_nei
    bAlc = b_low_temp @ W_lc
    cl = b_low_temp @ (W_lg_self + 0.2 * W_lr) + b_lg + 0.2 * b_lr

    z = jnp.zeros((OD, T), f32)
    Wm = jnp.concatenate([
        jnp.concatenate([Ah.T, z, Ch.T], axis=1),
        jnp.concatenate([Al.T, Alc.T, Cl.T], axis=1),
        jnp.concatenate([z, z, 2.0 * W_gr.T], axis=1),
    ], axis=0)
    P = jnp.stack(
        [bAh, ch, bAl, bAlc, cl, b_gr, g_h, be_h, g_l, be_l, g_g, be_g,
         W_att[:OD], W_att[OD:], jnp.full((OD,), b_att, f32),
         jnp.zeros((OD,), f32)], axis=1)

    aggp, caggp = _sc_aggregate(xp, srcr, dstr, csrcr, cdstr,
                                R, CW, base_split, caus_split)
    fused, high, low = _tc_dense(x2d, aggp, caggp, Wm, P,
                                 B, T, BT, OD, N, R, Nb)
    return (fused, high, low)
